# baseline (speedup 1.0000x reference)
import jax
import jax.numpy as jnp
from jax import lax
from jax.experimental import pallas as pl
from jax.experimental.pallas import tpu as pltpu
from jax.experimental.pallas import tpu_sc as plsc

S = 2048
D = 1024
E = 8
DFF = 4 * D
K = 2

A = S * K
BLK = 128
NB = A // BLK + E
R = NB * BLK
NW = 32
TPW = S // NW


_BT = 256
_NS = S // _BT


def _router_body(feat_ref, adj_ref, wlr_ref, blr_ref, wcg_ref, bcg_ref,
                 ti_ref, tw_ref, aux_ref, ls_ref, acc_ref):
    i = pl.program_id(0)

    @pl.when(i == 0)
    def _init():
        ls_ref[...] = (
            jnp.dot(feat_ref[...], wlr_ref[...],
                    preferred_element_type=jnp.float32) + blr_ref[...])
        acc_ref[...] = jnp.zeros_like(acc_ref)

    feat_blk = feat_ref[pl.ds(i * _BT, _BT), :]
    adj = adj_ref[...]
    neigh = jnp.dot(adj, ls_ref[...], preferred_element_type=jnp.float32)
    cnt = jnp.sum(adj, axis=1, keepdims=True) + 1e-8
    neigh = neigh / cnt
    g = jax.nn.sigmoid(
        jnp.dot(feat_blk, wcg_ref[...],
                preferred_element_type=jnp.float32) + bcg_ref[...])
    ls_blk = ls_ref[pl.ds(i * _BT, _BT), :]
    logits = g * ls_blk + (1.0 - g) * neigh

    m = jnp.max(logits, axis=-1, keepdims=True)
    ex = jnp.exp(logits - m)
    p = ex / jnp.sum(ex, axis=-1, keepdims=True)

    acc_ref[...] += jnp.sum(p, axis=0, keepdims=True)

    lane = jax.lax.broadcasted_iota(jnp.int32, (_BT, E), 1)
    big = jnp.int32(2 * E)
    p1 = jnp.max(p, axis=-1, keepdims=True)
    i1 = jnp.min(jnp.where(p == p1, lane, big), axis=-1, keepdims=True)
    pm = jnp.where(lane == i1, -jnp.inf, p)
    p2 = jnp.max(pm, axis=-1, keepdims=True)
    i2 = jnp.min(jnp.where(pm == p2, lane, big), axis=-1, keepdims=True)
    den = p1 + p2 + 1e-8
    w1 = p1 / den
    w2 = p2 / den

    ti_ref[...] = jnp.concatenate([i1, i2], axis=-1)
    tw_ref[...] = jnp.concatenate([w1, w2], axis=-1)

    @pl.when(i == _NS - 1)
    def _aux():
        u = acc_ref[...]
        imp = u / (jnp.sum(u) + 1e-8)
        mean = jnp.mean(imp)
        std = jnp.sqrt(jnp.mean((imp - mean) ** 2))
        aux_ref[...] = (std / (mean + 1e-8)).reshape(1, 1)


def _router(features, adjacency, W_lr, b_lr, W_cg, b_cg):
    return pl.pallas_call(
        _router_body,
        grid=(_NS,),
        in_specs=[
            pl.BlockSpec((S, D), lambda i: (0, 0)),
            pl.BlockSpec((_BT, S), lambda i: (i, 0)),
            pl.BlockSpec((D, E), lambda i: (0, 0)),
            pl.BlockSpec((1, E), lambda i: (0, 0)),
            pl.BlockSpec((D, 1), lambda i: (0, 0)),
            pl.BlockSpec((1, 1), lambda i: (0, 0)),
        ],
        out_specs=[
            pl.BlockSpec((_BT, K), lambda i: (i, 0)),
            pl.BlockSpec((_BT, K), lambda i: (i, 0)),
            pl.BlockSpec((1, 1), lambda i: (0, 0)),
        ],
        out_shape=[
            jax.ShapeDtypeStruct((S, K), jnp.int32),
            jax.ShapeDtypeStruct((S, K), jnp.float32),
            jax.ShapeDtypeStruct((1, 1), jnp.float32),
        ],
        scratch_shapes=[
            pltpu.VMEM((S, E), jnp.float32),
            pltpu.VMEM((1, E), jnp.float32),
        ],
        compiler_params=pltpu.CompilerParams(
            dimension_semantics=("arbitrary",)),
    )(features, adjacency, W_lr, b_lr.reshape(1, E), W_cg,
      b_cg.reshape(1, 1))


_TB = 256


def _slot_body(ti_ref, ps_ref, tk_ref, be_ref):
    lane = jax.lax.broadcasted_iota(jnp.int32, (1, 128), 1)
    lS = jax.lax.broadcasted_iota(jnp.int32, (S, 128), 1)
    e0 = ti_ref[:, 0:1]
    e1 = ti_ref[:, 1:2]
    oh0 = (lS == e0).astype(jnp.float32)
    oh1 = (lS == e1).astype(jnp.float32)

    c0 = jnp.sum(oh0, axis=0, keepdims=True)
    ctot = c0 + jnp.sum(oh1, axis=0, keepdims=True)

    nb = jnp.ceil(ctot / float(BLK))
    r128 = jax.lax.broadcasted_iota(jnp.int32, (128, 128), 0)
    c128 = jax.lax.broadcasted_iota(jnp.int32, (128, 128), 1)
    ut = (r128 <= c128).astype(jnp.float32)
    incl_blk = jnp.dot(nb, ut, preferred_element_type=jnp.float32)
    base_e = (incl_blk - nb) * float(BLK)

    zf = jnp.zeros((S, 1), jnp.float32)
    basea0 = zf
    basea1 = zf
    for e in range(E):
        sel = lane == e
        be_s = jnp.sum(jnp.where(sel, base_e, 0.0))
        c0_s = jnp.sum(jnp.where(sel, c0, 0.0))
        basea0 = basea0 + jnp.where(e0 == e, be_s, 0.0)
        basea1 = basea1 + jnp.where(e1 == e, be_s + c0_s, 0.0)

    rT = jax.lax.broadcasted_iota(jnp.int32, (_TB, _TB), 0)
    cT = jax.lax.broadcasted_iota(jnp.int32, (_TB, _TB), 1)
    tril = (rT >= cT).astype(jnp.float32)

    def excl_cumsum(oh):
        run = jnp.zeros((1, 128), jnp.float32)
        parts = []
        for b in range(S // _TB):
            blk = oh[b * _TB:(b + 1) * _TB, :]
            incl = jnp.dot(tril, blk, preferred_element_type=jnp.float32)
            parts.append(incl - blk + run)
            run = run + jnp.sum(blk, axis=0, keepdims=True)
        return jnp.concatenate(parts, axis=0)

    ex0 = excl_cumsum(oh0)
    ex1 = excl_cumsum(oh1)
    rank0 = jnp.sum(jnp.where(lS == e0, ex0, 0.0), axis=1, keepdims=True)
    rank1 = jnp.sum(jnp.where(lS == e1, ex1, 0.0), axis=1, keepdims=True)

    slot0 = (basea0 + rank0).astype(jnp.int32)
    slot1 = (basea1 + rank1).astype(jnp.int32)

    ps_ref[...] = jnp.concatenate([slot0, slot1], axis=-1)
    tk_ref[...] = jax.lax.broadcasted_iota(jnp.int32, (S, K), 0)

    acc = jnp.zeros((1, 128), jnp.int32)
    for e in range(E):
        incl_s = jnp.sum(jnp.where(lane == e, incl_blk, 0.0))
        acc = acc + (lane.astype(jnp.float32) >= incl_s).astype(jnp.int32)
    be_ref[...] = jnp.minimum(acc, E - 1)


def _slots(ti):
    return pl.pallas_call(
        _slot_body,
        out_shape=[
            jax.ShapeDtypeStruct((S, K), jnp.int32),
            jax.ShapeDtypeStruct((S, K), jnp.int32),
            jax.ShapeDtypeStruct((1, 128), jnp.int32),
        ],
    )(ti)


def _scatter_ids_body(psi, tk, wv, sid_hbm, wslot, s_v, t_v, w_v, sem):
    wid = lax.axis_index("s") * 2 + lax.axis_index("c")
    pltpu.sync_copy(psi.at[wid], s_v)
    pltpu.sync_copy(tk.at[wid], t_v)
    pltpu.sync_copy(wv.at[wid], w_v)
    ct = pltpu.async_copy(t_v.at[0], sid_hbm.at[s_v.at[0]], sem)
    cw = pltpu.async_copy(w_v.at[0], wslot.at[s_v.at[0]], sem)
    ct.wait()
    cw.wait()


def _scatter_ids(psi, tk, wv):
    mesh = plsc.VectorSubcoreMesh(core_axis_name="c", subcore_axis_name="s")
    return pl.kernel(
        _scatter_ids_body,
        out_type=[
            jax.ShapeDtypeStruct((R,), jnp.int32),
            jax.ShapeDtypeStruct((R,), jnp.float32),
        ],
        mesh=mesh,
        scratch_types=[
            pltpu.VMEM((1, 128), jnp.int32),
            pltpu.VMEM((1, 128), jnp.int32),
            pltpu.VMEM((1, 128), jnp.float32),
            pltpu.SemaphoreType.DMA,
        ],
    )(psi, tk, wv)


_GR = R // NW
_GC = 80


def _gather_body(feat_hbm, sid_hbm, g_hbm, idx_v, rows_v, sem):
    wid = lax.axis_index("s") * 2 + lax.axis_index("c")
    base = pl.multiple_of(wid * _GR, 8)
    pltpu.sync_copy(sid_hbm.at[pl.ds(base, _GR)], idx_v)
    for m in range(_GR // 16):
        v = idx_v[pl.ds(m * 16, 16)]
        idx_v[pl.ds(m * 16, 16)] = jnp.minimum(
            jnp.maximum(v, 0), S - 1)
    for c in range(_GR // _GC):
        pltpu.async_copy(feat_hbm.at[idx_v.at[pl.ds(c * _GC, _GC)]],
                         rows_v, sem).wait()
        pltpu.sync_copy(rows_v,
                        g_hbm.at[pl.ds(pl.multiple_of(base + c * _GC, 8),
                                       _GC)])


def _gather(feat, sid):
    mesh = plsc.VectorSubcoreMesh(core_axis_name="c", subcore_axis_name="s")
    return pl.kernel(
        _gather_body,
        out_type=jax.ShapeDtypeStruct((R, D), jnp.float32),
        mesh=mesh,
        scratch_types=[
            pltpu.VMEM((_GR,), jnp.int32),
            pltpu.VMEM((_GC, D), jnp.float32),
            pltpu.SemaphoreType.DMA,
        ],
    )(feat, sid)


def _ffn_body(be_ref, g_ref, wup_ref, bup_ref, wdn_ref, bdn_ref, ws_ref,
              y_ref):
    gb = g_ref[...].astype(jnp.bfloat16)
    up = (jnp.dot(gb, wup_ref[0],
                  preferred_element_type=jnp.float32) + bup_ref[0])
    h = jax.nn.gelu(up).astype(jnp.bfloat16)
    part = jnp.dot(h, wdn_ref[0], preferred_element_type=jnp.float32)
    y_ref[...] = (part + bdn_ref[0]) * ws_ref[...]


def _ffn_sparse(blocke, g, W_up, b_up, W_down, b_down, wslot):
    grid_spec = pltpu.PrefetchScalarGridSpec(
        num_scalar_prefetch=1,
        grid=(NB,),
        in_specs=[
            pl.BlockSpec((BLK, D), lambda b, se: (b, 0)),
            pl.BlockSpec((1, D, DFF), lambda b, se: (se[b], 0, 0)),
            pl.BlockSpec((1, 1, DFF), lambda b, se: (se[b], 0, 0)),
            pl.BlockSpec((1, DFF, D), lambda b, se: (se[b], 0, 0)),
            pl.BlockSpec((1, 1, D), lambda b, se: (se[b], 0, 0)),
            pl.BlockSpec((BLK, 1), lambda b, se: (b, 0)),
        ],
        out_specs=pl.BlockSpec((BLK, D), lambda b, se: (b, 0)),
    )
    return pl.pallas_call(
        _ffn_body,
        grid_spec=grid_spec,
        out_shape=jax.ShapeDtypeStruct((R, D), jnp.float32),
        compiler_params=pltpu.CompilerParams(
            dimension_semantics=("arbitrary",)),
    )(blocke, g, W_up.astype(jnp.bfloat16), b_up.reshape(E, 1, DFF),
      W_down.astype(jnp.bfloat16), b_down.reshape(E, 1, D), wslot)


_CC = TPW // 2


def _combine_body(y_hbm, pos_hbm, o_hbm, pos_v, rows_v, out_v, sem):
    wid = lax.axis_index("s") * 2 + lax.axis_index("c")
    base_t = pl.multiple_of(wid * TPW, 8)
    pltpu.sync_copy(pos_hbm.at[wid], pos_v)
    for c in range(2):
        pltpu.async_copy(y_hbm.at[pos_v.at[pl.ds(c * 2 * _CC, 2 * _CC)]],
                         rows_v, sem).wait()

        def body(i, _):
            for m in range(D // 16):
                sl = pl.ds(m * 16, 16)
                out_v[i, sl] = rows_v[2 * i, sl] + rows_v[2 * i + 1, sl]
            return 0

        lax.fori_loop(0, _CC, body, 0)
        pltpu.sync_copy(out_v,
                        o_hbm.at[pl.ds(pl.multiple_of(base_t + c * _CC, 8),
                                       _CC)])


def _combine(y, positions):
    mesh = plsc.VectorSubcoreMesh(core_axis_name="c", subcore_axis_name="s")
    return pl.kernel(
        _combine_body,
        out_type=jax.ShapeDtypeStruct((S, D), jnp.float32),
        mesh=mesh,
        scratch_types=[
            pltpu.VMEM((2 * TPW,), jnp.int32),
            pltpu.VMEM((2 * _CC, D), jnp.float32),
            pltpu.VMEM((_CC, D), jnp.float32),
            pltpu.SemaphoreType.DMA,
        ],
    )(y, positions)


def kernel(features, adjacency, W_lr, b_lr, W_cg, b_cg, W_up, b_up,
           W_down, b_down):
    feat = features.reshape(S, D)
    adj = adjacency.reshape(S, S)
    ti, tw, aux = _router(feat, adj, W_lr, b_lr, W_cg, b_cg)
    ps, tk, be = _slots(ti)
    sid, wslot = _scatter_ids(
        ps.reshape(NW, 1, 128),
        tk.reshape(NW, 1, 128),
        tw.reshape(NW, 1, 128),
    )
    g = _gather(feat, sid)
    y = _ffn_sparse(be[0, :NB], g, W_up, b_up, W_down,
                    b_down, wslot.reshape(R, 1))
    out = _combine(y, ps.reshape(NW, 2 * TPW))
    return (out.reshape(1, S, D), ti.reshape(1, S, K), aux.reshape(()))


# speedup vs baseline: 1.1582x; 1.1582x over previous
import jax
import jax.numpy as jnp
from jax import lax
from jax.experimental import pallas as pl
from jax.experimental.pallas import tpu as pltpu
from jax.experimental.pallas import tpu_sc as plsc

S = 2048
D = 1024
E = 8
DFF = 4 * D
K = 2

A = S * K
BLK = 128
NB = A // BLK + E
R = NB * BLK
NW = 32
TPW = S // NW


_BT = 256
_NS = S // _BT


def _router_body(feat_ref, adj_ref, wlr_ref, blr_ref, wcg_ref, bcg_ref,
                 ti_ref, tw_ref, aux_ref, ls_ref, acc_ref):
    i = pl.program_id(0)

    @pl.when(i == 0)
    def _init():
        ls_ref[...] = (
            jnp.dot(feat_ref[...], wlr_ref[...],
                    preferred_element_type=jnp.float32) + blr_ref[...])
        acc_ref[...] = jnp.zeros_like(acc_ref)

    feat_blk = feat_ref[pl.ds(i * _BT, _BT), :]
    adj = adj_ref[...]
    neigh = jnp.dot(adj, ls_ref[...], preferred_element_type=jnp.float32)
    cnt = jnp.sum(adj, axis=1, keepdims=True) + 1e-8
    neigh = neigh / cnt
    g = jax.nn.sigmoid(
        jnp.dot(feat_blk, wcg_ref[...],
                preferred_element_type=jnp.float32) + bcg_ref[...])
    ls_blk = ls_ref[pl.ds(i * _BT, _BT), :]
    logits = g * ls_blk + (1.0 - g) * neigh

    m = jnp.max(logits, axis=-1, keepdims=True)
    ex = jnp.exp(logits - m)
    p = ex / jnp.sum(ex, axis=-1, keepdims=True)

    acc_ref[...] += jnp.sum(p, axis=0, keepdims=True)

    lane = jax.lax.broadcasted_iota(jnp.int32, (_BT, E), 1)
    big = jnp.int32(2 * E)
    p1 = jnp.max(p, axis=-1, keepdims=True)
    i1 = jnp.min(jnp.where(p == p1, lane, big), axis=-1, keepdims=True)
    pm = jnp.where(lane == i1, -jnp.inf, p)
    p2 = jnp.max(pm, axis=-1, keepdims=True)
    i2 = jnp.min(jnp.where(pm == p2, lane, big), axis=-1, keepdims=True)
    den = p1 + p2 + 1e-8
    w1 = p1 / den
    w2 = p2 / den

    ti_ref[...] = jnp.concatenate([i1, i2], axis=-1)
    tw_ref[...] = jnp.concatenate([w1, w2], axis=-1)

    @pl.when(i == _NS - 1)
    def _aux():
        u = acc_ref[...]
        imp = u / (jnp.sum(u) + 1e-8)
        mean = jnp.mean(imp)
        std = jnp.sqrt(jnp.mean((imp - mean) ** 2))
        aux_ref[...] = (std / (mean + 1e-8)).reshape(1, 1)


def _router(features, adjacency, W_lr, b_lr, W_cg, b_cg):
    return pl.pallas_call(
        _router_body,
        grid=(_NS,),
        in_specs=[
            pl.BlockSpec((S, D), lambda i: (0, 0)),
            pl.BlockSpec((_BT, S), lambda i: (i, 0)),
            pl.BlockSpec((D, E), lambda i: (0, 0)),
            pl.BlockSpec((1, E), lambda i: (0, 0)),
            pl.BlockSpec((D, 1), lambda i: (0, 0)),
            pl.BlockSpec((1, 1), lambda i: (0, 0)),
        ],
        out_specs=[
            pl.BlockSpec((_BT, K), lambda i: (i, 0)),
            pl.BlockSpec((_BT, K), lambda i: (i, 0)),
            pl.BlockSpec((1, 1), lambda i: (0, 0)),
        ],
        out_shape=[
            jax.ShapeDtypeStruct((S, K), jnp.int32),
            jax.ShapeDtypeStruct((S, K), jnp.float32),
            jax.ShapeDtypeStruct((1, 1), jnp.float32),
        ],
        scratch_shapes=[
            pltpu.VMEM((S, E), jnp.float32),
            pltpu.VMEM((1, E), jnp.float32),
        ],
        compiler_params=pltpu.CompilerParams(
            dimension_semantics=("arbitrary",)),
    )(features, adjacency, W_lr, b_lr.reshape(1, E), W_cg,
      b_cg.reshape(1, 1))


_TB = 256


def _slot_body(ti_ref, ps_ref, s0_ref, s1_ref, be_ref):
    lane = jax.lax.broadcasted_iota(jnp.int32, (1, 128), 1)
    lS = jax.lax.broadcasted_iota(jnp.int32, (S, 128), 1)
    e0 = ti_ref[:, 0:1]
    e1 = ti_ref[:, 1:2]
    oh0 = (lS == e0).astype(jnp.float32)
    oh1 = (lS == e1).astype(jnp.float32)

    c0 = jnp.sum(oh0, axis=0, keepdims=True)
    ctot = c0 + jnp.sum(oh1, axis=0, keepdims=True)

    nb = jnp.ceil(ctot / float(BLK))
    r128 = jax.lax.broadcasted_iota(jnp.int32, (128, 128), 0)
    c128 = jax.lax.broadcasted_iota(jnp.int32, (128, 128), 1)
    ut = (r128 <= c128).astype(jnp.float32)
    incl_blk = jnp.dot(nb, ut, preferred_element_type=jnp.float32)
    base_e = (incl_blk - nb) * float(BLK)

    zf = jnp.zeros((S, 1), jnp.float32)
    basea0 = zf
    basea1 = zf
    for e in range(E):
        sel = lane == e
        be_s = jnp.sum(jnp.where(sel, base_e, 0.0))
        c0_s = jnp.sum(jnp.where(sel, c0, 0.0))
        basea0 = basea0 + jnp.where(e0 == e, be_s, 0.0)
        basea1 = basea1 + jnp.where(e1 == e, be_s + c0_s, 0.0)

    rT = jax.lax.broadcasted_iota(jnp.int32, (_TB, _TB), 0)
    cT = jax.lax.broadcasted_iota(jnp.int32, (_TB, _TB), 1)
    tril = (rT >= cT).astype(jnp.float32)

    def excl_cumsum(oh):
        run = jnp.zeros((1, 128), jnp.float32)
        parts = []
        for b in range(S // _TB):
            blk = oh[b * _TB:(b + 1) * _TB, :]
            incl = jnp.dot(tril, blk, preferred_element_type=jnp.float32)
            parts.append(incl - blk + run)
            run = run + jnp.sum(blk, axis=0, keepdims=True)
        return jnp.concatenate(parts, axis=0)

    ex0 = excl_cumsum(oh0)
    ex1 = excl_cumsum(oh1)
    rank0 = jnp.sum(jnp.where(lS == e0, ex0, 0.0), axis=1, keepdims=True)
    rank1 = jnp.sum(jnp.where(lS == e1, ex1, 0.0), axis=1, keepdims=True)

    slot0 = (basea0 + rank0).astype(jnp.int32)
    slot1 = (basea1 + rank1).astype(jnp.int32)

    ps_ref[...] = jnp.concatenate([slot0, slot1], axis=-1)
    p8 = jax.lax.broadcasted_iota(jnp.int32, (S, 8), 1)
    s0_ref[...] = slot0 * 8 + p8
    s1_ref[...] = slot1 * 8 + p8

    acc = jnp.zeros((1, 128), jnp.int32)
    for e in range(E):
        incl_s = jnp.sum(jnp.where(lane == e, incl_blk, 0.0))
        acc = acc + (lane.astype(jnp.float32) >= incl_s).astype(jnp.int32)
    be_ref[...] = jnp.minimum(acc, E - 1)


def _slots(ti):
    return pl.pallas_call(
        _slot_body,
        out_shape=[
            jax.ShapeDtypeStruct((S, K), jnp.int32),
            jax.ShapeDtypeStruct((S, 8), jnp.int32),
            jax.ShapeDtypeStruct((S, 8), jnp.int32),
            jax.ShapeDtypeStruct((1, 128), jnp.int32),
        ],
    )(ti)


_SR = TPW * 8


def _scatter_body(f8_hbm, s0_hbm, s1_hbm, psi, wv, g_hbm, wslot,
                  s0_v, s1_v, si_v, w_v, fv, sem):
    wid = lax.axis_index("s") * 2 + lax.axis_index("c")
    pltpu.sync_copy(s0_hbm.at[wid], s0_v)
    pltpu.sync_copy(s1_hbm.at[wid], s1_v)
    pltpu.sync_copy(psi.at[wid], si_v)
    pltpu.sync_copy(wv.at[wid], w_v)
    base = pl.multiple_of(wid * _SR, 8)
    pltpu.sync_copy(f8_hbm.at[pl.ds(base, _SR)], fv)
    cw = pltpu.async_copy(w_v.at[0], wslot.at[si_v.at[0]], sem)
    cps = []
    for c in range(_SR // 128):
        src = fv.at[pl.ds(c * 128, 128)]
        cps.append(pltpu.async_copy(src, g_hbm.at[s0_v.at[c]], sem))
        cps.append(pltpu.async_copy(src, g_hbm.at[s1_v.at[c]], sem))
    cw.wait()
    for cp in cps:
        cp.wait()


def _scatter(f8, s0, s1, psi, wv):
    mesh = plsc.VectorSubcoreMesh(core_axis_name="c", subcore_axis_name="s")
    return pl.kernel(
        _scatter_body,
        out_type=[
            jax.ShapeDtypeStruct((R * 8, 128), jnp.float32),
            jax.ShapeDtypeStruct((R,), jnp.float32),
        ],
        mesh=mesh,
        scratch_types=[
            pltpu.VMEM((4, 128), jnp.int32),
            pltpu.VMEM((4, 128), jnp.int32),
            pltpu.VMEM((1, 128), jnp.int32),
            pltpu.VMEM((1, 128), jnp.float32),
            pltpu.VMEM((_SR, 128), jnp.float32),
            pltpu.SemaphoreType.DMA,
        ],
    )(f8, s0, s1, psi, wv)


def _ffn_body(be_ref, g_ref, wup_ref, bup_ref, wdn_ref, bdn_ref, ws_ref,
              y_ref):
    gb = g_ref[...].astype(jnp.bfloat16)
    up = (jnp.dot(gb, wup_ref[0],
                  preferred_element_type=jnp.float32) + bup_ref[0])
    h = jax.nn.gelu(up).astype(jnp.bfloat16)
    part = jnp.dot(h, wdn_ref[0], preferred_element_type=jnp.float32)
    y_ref[...] = (part + bdn_ref[0]) * ws_ref[...]


def _ffn_sparse(blocke, g, W_up, b_up, W_down, b_down, wslot):
    grid_spec = pltpu.PrefetchScalarGridSpec(
        num_scalar_prefetch=1,
        grid=(NB,),
        in_specs=[
            pl.BlockSpec((BLK, D), lambda b, se: (b, 0)),
            pl.BlockSpec((1, D, DFF), lambda b, se: (se[b], 0, 0)),
            pl.BlockSpec((1, 1, DFF), lambda b, se: (se[b], 0, 0)),
            pl.BlockSpec((1, DFF, D), lambda b, se: (se[b], 0, 0)),
            pl.BlockSpec((1, 1, D), lambda b, se: (se[b], 0, 0)),
            pl.BlockSpec((BLK, 1), lambda b, se: (b, 0)),
        ],
        out_specs=pl.BlockSpec((BLK, D), lambda b, se: (b, 0)),
    )
    return pl.pallas_call(
        _ffn_body,
        grid_spec=grid_spec,
        out_shape=jax.ShapeDtypeStruct((R, D), jnp.float32),
        compiler_params=pltpu.CompilerParams(
            dimension_semantics=("arbitrary",)),
    )(blocke, g, W_up.astype(jnp.bfloat16), b_up.reshape(E, 1, DFF),
      W_down.astype(jnp.bfloat16), b_down.reshape(E, 1, D), wslot)


_CC = TPW // 2


def _combine_body(y_hbm, pos_hbm, o_hbm, pos_v, rows_v, out_v, sem):
    wid = lax.axis_index("s") * 2 + lax.axis_index("c")
    base_t = pl.multiple_of(wid * TPW, 8)
    pltpu.sync_copy(pos_hbm.at[wid], pos_v)
    for c in range(2):
        pltpu.async_copy(y_hbm.at[pos_v.at[pl.ds(c * 2 * _CC, 2 * _CC)]],
                         rows_v, sem).wait()

        def body(i, _):
            for m in range(D // 16):
                sl = pl.ds(m * 16, 16)
                out_v[i, sl] = rows_v[2 * i, sl] + rows_v[2 * i + 1, sl]
            return 0

        lax.fori_loop(0, _CC, body, 0)
        pltpu.sync_copy(out_v,
                        o_hbm.at[pl.ds(pl.multiple_of(base_t + c * _CC, 8),
                                       _CC)])


def _combine(y, positions):
    mesh = plsc.VectorSubcoreMesh(core_axis_name="c", subcore_axis_name="s")
    return pl.kernel(
        _combine_body,
        out_type=jax.ShapeDtypeStruct((S, D), jnp.float32),
        mesh=mesh,
        scratch_types=[
            pltpu.VMEM((2 * TPW,), jnp.int32),
            pltpu.VMEM((2 * _CC, D), jnp.float32),
            pltpu.VMEM((_CC, D), jnp.float32),
            pltpu.SemaphoreType.DMA,
        ],
    )(y, positions)


def kernel(features, adjacency, W_lr, b_lr, W_cg, b_cg, W_up, b_up,
           W_down, b_down):
    feat = features.reshape(S, D)
    adj = adjacency.reshape(S, S)
    ti, tw, aux = _router(feat, adj, W_lr, b_lr, W_cg, b_cg)
    ps, s0, s1, be = _slots(ti)
    g8, wslot = _scatter(
        feat.reshape(S * 8, 128),
        s0.reshape(NW, 4, 128),
        s1.reshape(NW, 4, 128),
        ps.reshape(NW, 1, 128),
        tw.reshape(NW, 1, 128),
    )
    y = _ffn_sparse(be[0, :NB], g8.reshape(R, D), W_up, b_up, W_down,
                    b_down, wslot.reshape(R, 1))
    out = _combine(y, ps.reshape(NW, 2 * TPW))
    return (out.reshape(1, S, D), ti.reshape(1, S, K), aux.reshape(()))


# speedup vs baseline: 1.2624x; 1.0899x over previous
import jax
import jax.numpy as jnp
from jax import lax
from jax.experimental import pallas as pl
from jax.experimental.pallas import tpu as pltpu
from jax.experimental.pallas import tpu_sc as plsc

S = 2048
D = 1024
E = 8
DFF = 4 * D
K = 2

A = S * K
BLK = 128
NB = A // BLK + E
R = NB * BLK
NW = 32
TPW = S // NW


_BT = 256
_NS = S // _BT


def _router_body(feat_ref, adj_ref, wlr_ref, blr_ref, wcg_ref, bcg_ref,
                 ti_ref, tw_ref, aux_ref, ls_ref, acc_ref):
    i = pl.program_id(0)

    @pl.when(i == 0)
    def _init():
        ls_ref[...] = (
            jnp.dot(feat_ref[...], wlr_ref[...],
                    preferred_element_type=jnp.float32) + blr_ref[...])
        acc_ref[...] = jnp.zeros_like(acc_ref)

    feat_blk = feat_ref[pl.ds(i * _BT, _BT), :]
    adj = adj_ref[...]
    neigh = jnp.dot(adj, ls_ref[...], preferred_element_type=jnp.float32)
    cnt = jnp.sum(adj, axis=1, keepdims=True) + 1e-8
    neigh = neigh / cnt
    g = jax.nn.sigmoid(
        jnp.dot(feat_blk, wcg_ref[...],
                preferred_element_type=jnp.float32) + bcg_ref[...])
    ls_blk = ls_ref[pl.ds(i * _BT, _BT), :]
    logits = g * ls_blk + (1.0 - g) * neigh

    m = jnp.max(logits, axis=-1, keepdims=True)
    ex = jnp.exp(logits - m)
    p = ex / jnp.sum(ex, axis=-1, keepdims=True)

    acc_ref[...] += jnp.sum(p, axis=0, keepdims=True)

    lane = jax.lax.broadcasted_iota(jnp.int32, (_BT, E), 1)
    big = jnp.int32(2 * E)
    p1 = jnp.max(p, axis=-1, keepdims=True)
    i1 = jnp.min(jnp.where(p == p1, lane, big), axis=-1, keepdims=True)
    pm = jnp.where(lane == i1, -jnp.inf, p)
    p2 = jnp.max(pm, axis=-1, keepdims=True)
    i2 = jnp.min(jnp.where(pm == p2, lane, big), axis=-1, keepdims=True)
    den = p1 + p2 + 1e-8
    w1 = p1 / den
    w2 = p2 / den

    ti_ref[...] = jnp.concatenate([i1, i2], axis=-1)
    tw_ref[...] = jnp.concatenate([w1, w2], axis=-1)

    @pl.when(i == _NS - 1)
    def _aux():
        u = acc_ref[...]
        imp = u / (jnp.sum(u) + 1e-8)
        mean = jnp.mean(imp)
        std = jnp.sqrt(jnp.mean((imp - mean) ** 2))
        aux_ref[...] = (std / (mean + 1e-8)).reshape(1, 1)


def _router(features, adjacency, W_lr, b_lr, W_cg, b_cg):
    return pl.pallas_call(
        _router_body,
        grid=(_NS,),
        in_specs=[
            pl.BlockSpec((S, D), lambda i: (0, 0)),
            pl.BlockSpec((_BT, S), lambda i: (i, 0)),
            pl.BlockSpec((D, E), lambda i: (0, 0)),
            pl.BlockSpec((1, E), lambda i: (0, 0)),
            pl.BlockSpec((D, 1), lambda i: (0, 0)),
            pl.BlockSpec((1, 1), lambda i: (0, 0)),
        ],
        out_specs=[
            pl.BlockSpec((_BT, K), lambda i: (i, 0)),
            pl.BlockSpec((_BT, K), lambda i: (i, 0)),
            pl.BlockSpec((1, 1), lambda i: (0, 0)),
        ],
        out_shape=[
            jax.ShapeDtypeStruct((S, K), jnp.int32),
            jax.ShapeDtypeStruct((S, K), jnp.float32),
            jax.ShapeDtypeStruct((1, 1), jnp.float32),
        ],
        scratch_shapes=[
            pltpu.VMEM((S, E), jnp.float32),
            pltpu.VMEM((1, E), jnp.float32),
        ],
        compiler_params=pltpu.CompilerParams(
            dimension_semantics=("arbitrary",)),
    )(features, adjacency, W_lr, b_lr.reshape(1, E), W_cg,
      b_cg.reshape(1, 1))


_TB = 256


def _slot_body(ti_ref, ps_ref, s0_ref, s1_ref, be_ref):
    lane = jax.lax.broadcasted_iota(jnp.int32, (1, 128), 1)
    lS = jax.lax.broadcasted_iota(jnp.int32, (S, 128), 1)
    e0 = ti_ref[:, 0:1]
    e1 = ti_ref[:, 1:2]
    oh0 = (lS == e0).astype(jnp.float32)
    oh1 = (lS == e1).astype(jnp.float32)

    c0 = jnp.sum(oh0, axis=0, keepdims=True)
    ctot = c0 + jnp.sum(oh1, axis=0, keepdims=True)

    nb = jnp.ceil(ctot / float(BLK))
    r128 = jax.lax.broadcasted_iota(jnp.int32, (128, 128), 0)
    c128 = jax.lax.broadcasted_iota(jnp.int32, (128, 128), 1)
    ut = (r128 <= c128).astype(jnp.float32)
    incl_blk = jnp.dot(nb, ut, preferred_element_type=jnp.float32)
    base_e = (incl_blk - nb) * float(BLK)

    zf = jnp.zeros((S, 1), jnp.float32)
    basea0 = zf
    basea1 = zf
    for e in range(E):
        sel = lane == e
        be_s = jnp.sum(jnp.where(sel, base_e, 0.0))
        c0_s = jnp.sum(jnp.where(sel, c0, 0.0))
        basea0 = basea0 + jnp.where(e0 == e, be_s, 0.0)
        basea1 = basea1 + jnp.where(e1 == e, be_s + c0_s, 0.0)

    rT = jax.lax.broadcasted_iota(jnp.int32, (_TB, _TB), 0)
    cT = jax.lax.broadcasted_iota(jnp.int32, (_TB, _TB), 1)
    tril = (rT >= cT).astype(jnp.float32)

    def excl_cumsum(oh):
        run = jnp.zeros((1, 128), jnp.float32)
        parts = []
        for b in range(S // _TB):
            blk = oh[b * _TB:(b + 1) * _TB, :]
            incl = jnp.dot(tril, blk, preferred_element_type=jnp.float32)
            parts.append(incl - blk + run)
            run = run + jnp.sum(blk, axis=0, keepdims=True)
        return jnp.concatenate(parts, axis=0)

    ex0 = excl_cumsum(oh0)
    ex1 = excl_cumsum(oh1)
    rank0 = jnp.sum(jnp.where(lS == e0, ex0, 0.0), axis=1, keepdims=True)
    rank1 = jnp.sum(jnp.where(lS == e1, ex1, 0.0), axis=1, keepdims=True)

    slot0 = (basea0 + rank0).astype(jnp.int32)
    slot1 = (basea1 + rank1).astype(jnp.int32)

    ps_ref[...] = jnp.concatenate([slot0, slot1], axis=-1)
    p8 = jax.lax.broadcasted_iota(jnp.int32, (S, 8), 1)
    s0_ref[...] = slot0 * 8 + p8
    s1_ref[...] = slot1 * 8 + p8

    acc = jnp.zeros((1, 128), jnp.int32)
    for e in range(E):
        incl_s = jnp.sum(jnp.where(lane == e, incl_blk, 0.0))
        acc = acc + (lane.astype(jnp.float32) >= incl_s).astype(jnp.int32)
    be_ref[...] = jnp.minimum(acc, E - 1)


def _slots(ti):
    return pl.pallas_call(
        _slot_body,
        out_shape=[
            jax.ShapeDtypeStruct((S, K), jnp.int32),
            jax.ShapeDtypeStruct((S, 8), jnp.int32),
            jax.ShapeDtypeStruct((S, 8), jnp.int32),
            jax.ShapeDtypeStruct((1, 128), jnp.int32),
        ],
    )(ti)


def _scatter_body(feat_hbm, s2_hbm, psi, wv, g_hbm, wslot,
                  s2_v, si_v, w_v, fv, sem):
    wid = lax.axis_index("s") * 2 + lax.axis_index("c")
    pltpu.sync_copy(s2_hbm.at[wid], s2_v)
    pltpu.sync_copy(psi.at[wid], si_v)
    pltpu.sync_copy(wv.at[wid], w_v)
    base = pl.multiple_of(wid * TPW, 8)
    pltpu.sync_copy(feat_hbm.at[pl.ds(base, TPW)], fv)
    cw = pltpu.async_copy(w_v.at[0], wslot.at[si_v.at[0]], sem)
    c0 = pltpu.async_copy(fv, g_hbm.at[s2_v.at[0]], sem)
    c1 = pltpu.async_copy(fv, g_hbm.at[s2_v.at[1]], sem)
    cw.wait()
    c0.wait()
    c1.wait()


def _scatter(feat, s2, psi, wv):
    mesh = plsc.VectorSubcoreMesh(core_axis_name="c", subcore_axis_name="s")
    return pl.kernel(
        _scatter_body,
        out_type=[
            jax.ShapeDtypeStruct((R, D), jnp.float32),
            jax.ShapeDtypeStruct((R,), jnp.float32),
        ],
        mesh=mesh,
        scratch_types=[
            pltpu.VMEM((K, TPW), jnp.int32),
            pltpu.VMEM((1, 128), jnp.int32),
            pltpu.VMEM((1, 128), jnp.float32),
            pltpu.VMEM((TPW, D), jnp.float32),
            pltpu.SemaphoreType.DMA,
        ],
    )(feat, s2, psi, wv)


def _ffn_body(be_ref, g_ref, wup_ref, bup_ref, wdn_ref, bdn_ref, ws_ref,
              y_ref):
    gb = g_ref[...].astype(jnp.bfloat16)
    up = (jnp.dot(gb, wup_ref[0],
                  preferred_element_type=jnp.float32) + bup_ref[0])
    h = jax.nn.gelu(up).astype(jnp.bfloat16)
    part = jnp.dot(h, wdn_ref[0], preferred_element_type=jnp.float32)
    y_ref[...] = (part + bdn_ref[0]) * ws_ref[...]


def _ffn_sparse(blocke, g, W_up, b_up, W_down, b_down, wslot):
    grid_spec = pltpu.PrefetchScalarGridSpec(
        num_scalar_prefetch=1,
        grid=(NB,),
        in_specs=[
            pl.BlockSpec((BLK, D), lambda b, se: (b, 0)),
            pl.BlockSpec((1, D, DFF), lambda b, se: (se[b], 0, 0)),
            pl.BlockSpec((1, 1, DFF), lambda b, se: (se[b], 0, 0)),
            pl.BlockSpec((1, DFF, D), lambda b, se: (se[b], 0, 0)),
            pl.BlockSpec((1, 1, D), lambda b, se: (se[b], 0, 0)),
            pl.BlockSpec((BLK, 1), lambda b, se: (b, 0)),
        ],
        out_specs=pl.BlockSpec((BLK, D), lambda b, se: (b, 0)),
    )
    return pl.pallas_call(
        _ffn_body,
        grid_spec=grid_spec,
        out_shape=jax.ShapeDtypeStruct((R, D), jnp.float32),
        compiler_params=pltpu.CompilerParams(
            dimension_semantics=("arbitrary",)),
    )(blocke, g, W_up.astype(jnp.bfloat16), b_up.reshape(E, 1, DFF),
      W_down.astype(jnp.bfloat16), b_down.reshape(E, 1, D), wslot)


_CC = TPW // 2


def _combine_body(y_hbm, pos_hbm, o_hbm, pos_v, rows_v, out_v, sem):
    wid = lax.axis_index("s") * 2 + lax.axis_index("c")
    base_t = pl.multiple_of(wid * TPW, 8)
    pltpu.sync_copy(pos_hbm.at[wid], pos_v)
    for c in range(2):
        pltpu.async_copy(y_hbm.at[pos_v.at[pl.ds(c * 2 * _CC, 2 * _CC)]],
                         rows_v, sem).wait()

        def body(i, _):
            for m in range(D // 16):
                sl = pl.ds(m * 16, 16)
                out_v[i, sl] = rows_v[2 * i, sl] + rows_v[2 * i + 1, sl]
            return 0

        lax.fori_loop(0, _CC, body, 0)
        pltpu.sync_copy(out_v,
                        o_hbm.at[pl.ds(pl.multiple_of(base_t + c * _CC, 8),
                                       _CC)])


def _combine(y, positions):
    mesh = plsc.VectorSubcoreMesh(core_axis_name="c", subcore_axis_name="s")
    return pl.kernel(
        _combine_body,
        out_type=jax.ShapeDtypeStruct((S, D), jnp.float32),
        mesh=mesh,
        scratch_types=[
            pltpu.VMEM((2 * TPW,), jnp.int32),
            pltpu.VMEM((2 * _CC, D), jnp.float32),
            pltpu.VMEM((_CC, D), jnp.float32),
            pltpu.SemaphoreType.DMA,
        ],
    )(y, positions)


def kernel(features, adjacency, W_lr, b_lr, W_cg, b_cg, W_up, b_up,
           W_down, b_down):
    feat = features.reshape(S, D)
    adj = adjacency.reshape(S, S)
    ti, tw, aux = _router(feat, adj, W_lr, b_lr, W_cg, b_cg)
    ps, s0, s1, be = _slots(ti)
    g, wslot = _scatter(
        feat,
        jnp.transpose(ps.reshape(NW, TPW, K), (0, 2, 1)),
        ps.reshape(NW, 1, 128),
        tw.reshape(NW, 1, 128),
    )
    y = _ffn_sparse(be[0, :NB], g, W_up, b_up, W_down,
                    b_down, wslot.reshape(R, 1))
    out = _combine(y, ps.reshape(NW, 2 * TPW))
    return (out.reshape(1, S, D), ti.reshape(1, S, K), aux.reshape(()))


# speedup vs baseline: 1.2833x; 1.0166x over previous
import jax
import jax.numpy as jnp
from jax import lax
from jax.experimental import pallas as pl
from jax.experimental.pallas import tpu as pltpu
from jax.experimental.pallas import tpu_sc as plsc

S = 2048
D = 1024
E = 8
DFF = 4 * D
K = 2

A = S * K
BLK = 128
NB = A // BLK + E
R = NB * BLK
NW = 32
TPW = S // NW


_BT = 256
_NS = S // _BT


def _router_body(feat_ref, adj_ref, wlr_ref, blr_ref, wcg_ref, bcg_ref,
                 ti_ref, tw_ref, aux_ref, ls_ref, acc_ref):
    i = pl.program_id(0)

    @pl.when(i == 0)
    def _init():
        ls_ref[...] = (
            jnp.dot(feat_ref[...], wlr_ref[...],
                    preferred_element_type=jnp.float32) + blr_ref[...])
        acc_ref[...] = jnp.zeros_like(acc_ref)

    feat_blk = feat_ref[pl.ds(i * _BT, _BT), :]
    adj = adj_ref[...]
    neigh = jnp.dot(adj, ls_ref[...], preferred_element_type=jnp.float32)
    cnt = jnp.sum(adj, axis=1, keepdims=True) + 1e-8
    neigh = neigh / cnt
    g = jax.nn.sigmoid(
        jnp.dot(feat_blk, wcg_ref[...],
                preferred_element_type=jnp.float32) + bcg_ref[...])
    ls_blk = ls_ref[pl.ds(i * _BT, _BT), :]
    logits = g * ls_blk + (1.0 - g) * neigh

    m = jnp.max(logits, axis=-1, keepdims=True)
    ex = jnp.exp(logits - m)
    p = ex / jnp.sum(ex, axis=-1, keepdims=True)

    acc_ref[...] += jnp.sum(p, axis=0, keepdims=True)

    lane = jax.lax.broadcasted_iota(jnp.int32, (_BT, E), 1)
    big = jnp.int32(2 * E)
    p1 = jnp.max(p, axis=-1, keepdims=True)
    i1 = jnp.min(jnp.where(p == p1, lane, big), axis=-1, keepdims=True)
    pm = jnp.where(lane == i1, -jnp.inf, p)
    p2 = jnp.max(pm, axis=-1, keepdims=True)
    i2 = jnp.min(jnp.where(pm == p2, lane, big), axis=-1, keepdims=True)
    den = p1 + p2 + 1e-8
    w1 = p1 / den
    w2 = p2 / den

    ti_ref[...] = jnp.concatenate([i1, i2], axis=-1)
    tw_ref[...] = jnp.concatenate([w1, w2], axis=-1)

    @pl.when(i == _NS - 1)
    def _aux():
        u = acc_ref[...]
        imp = u / (jnp.sum(u) + 1e-8)
        mean = jnp.mean(imp)
        std = jnp.sqrt(jnp.mean((imp - mean) ** 2))
        aux_ref[...] = (std / (mean + 1e-8)).reshape(1, 1)


def _router(features, adjacency, W_lr, b_lr, W_cg, b_cg):
    return pl.pallas_call(
        _router_body,
        grid=(_NS,),
        in_specs=[
            pl.BlockSpec((S, D), lambda i: (0, 0)),
            pl.BlockSpec((_BT, S), lambda i: (i, 0)),
            pl.BlockSpec((D, E), lambda i: (0, 0)),
            pl.BlockSpec((1, E), lambda i: (0, 0)),
            pl.BlockSpec((D, 1), lambda i: (0, 0)),
            pl.BlockSpec((1, 1), lambda i: (0, 0)),
        ],
        out_specs=[
            pl.BlockSpec((_BT, K), lambda i: (i, 0)),
            pl.BlockSpec((_BT, K), lambda i: (i, 0)),
            pl.BlockSpec((1, 1), lambda i: (0, 0)),
        ],
        out_shape=[
            jax.ShapeDtypeStruct((S, K), jnp.int32),
            jax.ShapeDtypeStruct((S, K), jnp.float32),
            jax.ShapeDtypeStruct((1, 1), jnp.float32),
        ],
        scratch_shapes=[
            pltpu.VMEM((S, E), jnp.float32),
            pltpu.VMEM((1, E), jnp.float32),
        ],
        compiler_params=pltpu.CompilerParams(
            dimension_semantics=("arbitrary",)),
    )(features, adjacency, W_lr, b_lr.reshape(1, E), W_cg,
      b_cg.reshape(1, 1))


_TB = 256


def _slot_body(ti_ref, ps_ref, be_ref):
    lane = jax.lax.broadcasted_iota(jnp.int32, (1, 128), 1)
    lS = jax.lax.broadcasted_iota(jnp.int32, (S, 128), 1)
    e0 = ti_ref[:, 0:1]
    e1 = ti_ref[:, 1:2]
    oh0 = (lS == e0).astype(jnp.float32)
    oh1 = (lS == e1).astype(jnp.float32)

    c0 = jnp.sum(oh0, axis=0, keepdims=True)
    ctot = c0 + jnp.sum(oh1, axis=0, keepdims=True)

    nb = jnp.ceil(ctot / float(BLK))
    r128 = jax.lax.broadcasted_iota(jnp.int32, (128, 128), 0)
    c128 = jax.lax.broadcasted_iota(jnp.int32, (128, 128), 1)
    ut = (r128 <= c128).astype(jnp.float32)
    incl_blk = jnp.dot(nb, ut, preferred_element_type=jnp.float32)
    base_e = (incl_blk - nb) * float(BLK)

    zf = jnp.zeros((S, 1), jnp.float32)
    basea0 = zf
    basea1 = zf
    for e in range(E):
        sel = lane == e
        be_s = jnp.sum(jnp.where(sel, base_e, 0.0))
        c0_s = jnp.sum(jnp.where(sel, c0, 0.0))
        basea0 = basea0 + jnp.where(e0 == e, be_s, 0.0)
        basea1 = basea1 + jnp.where(e1 == e, be_s + c0_s, 0.0)

    rT = jax.lax.broadcasted_iota(jnp.int32, (_TB, _TB), 0)
    cT = jax.lax.broadcasted_iota(jnp.int32, (_TB, _TB), 1)
    tril = (rT >= cT).astype(jnp.float32)

    def excl_cumsum(oh):
        run = jnp.zeros((1, 128), jnp.float32)
        parts = []
        for b in range(S // _TB):
            blk = oh[b * _TB:(b + 1) * _TB, :]
            incl = jnp.dot(tril, blk, preferred_element_type=jnp.float32)
            parts.append(incl - blk + run)
            run = run + jnp.sum(blk, axis=0, keepdims=True)
        return jnp.concatenate(parts, axis=0)

    ex0 = excl_cumsum(oh0)
    ex1 = excl_cumsum(oh1)
    rank0 = jnp.sum(jnp.where(lS == e0, ex0, 0.0), axis=1, keepdims=True)
    rank1 = jnp.sum(jnp.where(lS == e1, ex1, 0.0), axis=1, keepdims=True)

    slot0 = (basea0 + rank0).astype(jnp.int32)
    slot1 = (basea1 + rank1).astype(jnp.int32)

    ps_ref[...] = jnp.concatenate([slot0, slot1], axis=-1)

    acc = jnp.zeros((1, 128), jnp.int32)
    for e in range(E):
        incl_s = jnp.sum(jnp.where(lane == e, incl_blk, 0.0))
        acc = acc + (lane.astype(jnp.float32) >= incl_s).astype(jnp.int32)
    be_ref[...] = jnp.minimum(acc, E - 1)


def _slots(ti):
    return pl.pallas_call(
        _slot_body,
        out_shape=[
            jax.ShapeDtypeStruct((S, K), jnp.int32),
            jax.ShapeDtypeStruct((1, 128), jnp.int32),
        ],
    )(ti)


def _scatter_body(feat_hbm, s2_hbm, g_hbm, s2_v, fv, sem):
    wid = lax.axis_index("s") * 2 + lax.axis_index("c")
    pltpu.sync_copy(s2_hbm.at[wid], s2_v)
    base = pl.multiple_of(wid * TPW, 8)
    pltpu.sync_copy(feat_hbm.at[pl.ds(base, TPW)], fv)
    c0 = pltpu.async_copy(fv, g_hbm.at[s2_v.at[0]], sem)
    c1 = pltpu.async_copy(fv, g_hbm.at[s2_v.at[1]], sem)
    c0.wait()
    c1.wait()


def _scatter(feat, s2):
    mesh = plsc.VectorSubcoreMesh(core_axis_name="c", subcore_axis_name="s")
    return pl.kernel(
        _scatter_body,
        out_type=jax.ShapeDtypeStruct((R, D), jnp.float32),
        mesh=mesh,
        scratch_types=[
            pltpu.VMEM((K, TPW), jnp.int32),
            pltpu.VMEM((TPW, D), jnp.float32),
            pltpu.SemaphoreType.DMA,
        ],
    )(feat, s2)


def _ffn_body(be_ref, g_ref, wup_ref, bup_ref, wdn_ref, bdn_ref, y_ref):
    gb = g_ref[...].astype(jnp.bfloat16)
    up = (jnp.dot(gb, wup_ref[0],
                  preferred_element_type=jnp.float32) + bup_ref[0])
    h = jax.nn.gelu(up).astype(jnp.bfloat16)
    part = jnp.dot(h, wdn_ref[0], preferred_element_type=jnp.float32)
    y_ref[...] = part + bdn_ref[0]


def _ffn_sparse(blocke, g, W_up, b_up, W_down, b_down):
    grid_spec = pltpu.PrefetchScalarGridSpec(
        num_scalar_prefetch=1,
        grid=(NB,),
        in_specs=[
            pl.BlockSpec((BLK, D), lambda b, se: (b, 0)),
            pl.BlockSpec((1, D, DFF), lambda b, se: (se[b], 0, 0)),
            pl.BlockSpec((1, 1, DFF), lambda b, se: (se[b], 0, 0)),
            pl.BlockSpec((1, DFF, D), lambda b, se: (se[b], 0, 0)),
            pl.BlockSpec((1, 1, D), lambda b, se: (se[b], 0, 0)),
        ],
        out_specs=pl.BlockSpec((BLK, D), lambda b, se: (b, 0)),
    )
    return pl.pallas_call(
        _ffn_body,
        grid_spec=grid_spec,
        out_shape=jax.ShapeDtypeStruct((R, D), jnp.float32),
        compiler_params=pltpu.CompilerParams(
            dimension_semantics=("arbitrary",)),
    )(blocke, g, W_up.astype(jnp.bfloat16), b_up.reshape(E, 1, DFF),
      W_down.astype(jnp.bfloat16), b_down.reshape(E, 1, D))


_CC = TPW // 2


def _combine_body(y_hbm, pos_hbm, w_hbm, o_hbm, pos_v, wb_v, rows_v,
                  out_v, sem):
    wid = lax.axis_index("s") * 2 + lax.axis_index("c")
    base_t = pl.multiple_of(wid * TPW, 8)
    pltpu.sync_copy(pos_hbm.at[wid], pos_v)
    pltpu.sync_copy(w_hbm.at[wid], wb_v)
    for c in range(2):
        pltpu.async_copy(y_hbm.at[pos_v.at[pl.ds(c * 2 * _CC, 2 * _CC)]],
                         rows_v, sem).wait()

        def body(i, _):
            j = c * 2 * _CC + 2 * i
            w0 = wb_v[j, :]
            w1 = wb_v[j + 1, :]
            for m in range(D // 16):
                sl = pl.ds(m * 16, 16)
                out_v[i, sl] = (rows_v[2 * i, sl] * w0 +
                                rows_v[2 * i + 1, sl] * w1)
            return 0

        lax.fori_loop(0, _CC, body, 0)
        pltpu.sync_copy(out_v,
                        o_hbm.at[pl.ds(pl.multiple_of(base_t + c * _CC, 8),
                                       _CC)])


def _combine(y, positions, weights):
    mesh = plsc.VectorSubcoreMesh(core_axis_name="c", subcore_axis_name="s")
    return pl.kernel(
        _combine_body,
        out_type=jax.ShapeDtypeStruct((S, D), jnp.float32),
        mesh=mesh,
        scratch_types=[
            pltpu.VMEM((2 * TPW,), jnp.int32),
            pltpu.VMEM((2 * TPW, 16), jnp.float32),
            pltpu.VMEM((2 * _CC, D), jnp.float32),
            pltpu.VMEM((_CC, D), jnp.float32),
            pltpu.SemaphoreType.DMA,
        ],
    )(y, positions, weights)


def kernel(features, adjacency, W_lr, b_lr, W_cg, b_cg, W_up, b_up,
           W_down, b_down):
    feat = features.reshape(S, D)
    adj = adjacency.reshape(S, S)
    ti, tw, aux = _router(feat, adj, W_lr, b_lr, W_cg, b_cg)
    ps, be = _slots(ti)
    g = _scatter(
        feat,
        jnp.transpose(ps.reshape(NW, TPW, K), (0, 2, 1)),
    )
    y = _ffn_sparse(be[0, :NB], g, W_up, b_up, W_down, b_down)
    twb = jnp.broadcast_to(tw.reshape(A, 1), (A, 16))
    out = _combine(y, ps.reshape(NW, 2 * TPW),
                   twb.reshape(NW, 2 * TPW, 16))
    return (out.reshape(1, S, D), ti.reshape(1, S, K), aux.reshape(()))
